# Initial kernel scaffold; baseline (speedup 1.0000x reference)
import jax
import jax.numpy as jnp
from jax.experimental import pallas as pl


def kernel(views_1, views_2, img):
    raise NotImplementedError("write your pallas kernel here")


# speedup vs baseline: 7.4543x; 7.4543x over previous
import functools

import numpy as np
import jax
import jax.numpy as jnp
from jax import lax
from jax.experimental import pallas as pl
from jax.experimental.pallas import tpu as pltpu
from jax.experimental.pallas import tpu_sc as plsc

_B, _C, _H, _W = 8, 96, 224, 224
_GS = 8
_NG = 27
_P = _NG * _NG
_NEG = 256
_PIX = _H * _W
_ROWW = 128
_PPAD = 768
_GROUPS = _B * _P
_GTILE = _GROUPS // _NG
_NPAIR = _GROUPS * _NEG
_NW = 32
_CHUNK = 128
_TP = 512
_RGBC = float(0.2 / np.sqrt(3.0))
_EPS = 1e-8


@functools.lru_cache(maxsize=1)
def _host_constants():
    rng = np.random.RandomState(0)
    starts = []
    qidx = np.empty((_B, _P, _NEG), np.int32)
    eucw = np.empty((_B, _P, _NEG), np.float32)
    max_euc = np.sqrt(float((_H - 1) ** 2 + (_W - 1) ** 2))
    base = np.arange(0, _H - _GS, _GS)
    for b in range(_B):
        si = int(rng.choice(_GS, 1)[0])
        sj = int(rng.choice(_GS, 1)[0])
        starts.append((si, sj))
        ic, jc = np.meshgrid(base, base, indexing="ij")
        ic = ic + si
        jc = jc + sj
        neg_i = rng.randint(0, _H, size=(_P, _NEG))
        neg_j = rng.randint(0, _W, size=(_P, _NEG))
        qidx[b] = (b * _PIX + neg_i * _W + neg_j).astype(np.int32)
        coords = np.stack([ic.reshape(_P), jc.reshape(_P)], 0).astype(np.float32)
        negs = np.stack([neg_i, neg_j], 0).astype(np.float32)
        euc = np.linalg.norm(coords[:, :, None] - negs, axis=0) / max_euc
        eucw[b] = (euc * 0.8).astype(np.float32)
    return starts, qidx.reshape(-1), eucw.reshape(_GTILE, _NG, _NEG)


def _prep_table_body(v2_ref, img_ref, out_ref):
    x = v2_ref[0]
    out_ref[0, :, 0:_C] = x.T
    nrm = jnp.sqrt(jnp.sum(x * x, axis=0))
    out_ref[0, :, _C:_C + 1] = nrm[:, None]
    out_ref[0, :, _C + 1:_C + 4] = img_ref[...].T
    out_ref[0, :, _C + 4:_ROWW] = jnp.zeros((_TP, _ROWW - _C - 4), jnp.float32)


def _prep_table(v2, img0):
    out = pl.pallas_call(
        _prep_table_body,
        grid=(_B, _PIX // _TP),
        in_specs=[
            pl.BlockSpec((1, _C, _TP), lambda b, t: (b, 0, t)),
            pl.BlockSpec((3, _TP), lambda b, t: (0, t)),
        ],
        out_specs=pl.BlockSpec((1, _TP, _ROWW), lambda b, t: (b, t, 0)),
        out_shape=jax.ShapeDtypeStruct((_B, _PIX, _ROWW), jnp.float32),
    )(v2, img0)
    return out.reshape(_B * _PIX, _ROWW)


def _prep_z1_body(z_ref, rgb_ref, out_ref, nsq_ref):
    for b in range(_B):
        x = z_ref[b]
        out_ref[b, :, 0:_C] = x.T
        nsq = jnp.sum(x * x, axis=0)
        out_ref[b, :, _C:_C + 1] = jnp.sqrt(nsq)[:, None]
        out_ref[b, :, _C + 1:_C + 4] = rgb_ref[b].T
        out_ref[b, :, _C + 4:_ROWW] = jnp.zeros((_PPAD, _ROWW - _C - 4), jnp.float32)
        nsq_ref[b:b + 1, :] = nsq[None, :]


def _prep_z1(z1g, rgbg):
    return pl.pallas_call(
        _prep_z1_body,
        out_shape=(
            jax.ShapeDtypeStruct((_B, _PPAD, _ROWW), jnp.float32),
            jax.ShapeDtypeStruct((_B, _PPAD), jnp.float32),
        ),
    )(z1g, rgbg)


def _sc_main(table, z1flat, qidx):
    gpw = _GROUPS // _NW
    rem = _GROUPS % _NW
    mesh = plsc.VectorSubcoreMesh(core_axis_name="c", subcore_axis_name="s")

    @functools.partial(
        pl.kernel,
        mesh=mesh,
        out_type=(
            jax.ShapeDtypeStruct((_NPAIR, 16), jnp.float32),
            jax.ShapeDtypeStruct((_NPAIR, 16), jnp.float32),
        ),
        scratch_types=[
            pltpu.VMEM((_CHUNK,), jnp.int32),
            pltpu.VMEM((_CHUNK, _ROWW), jnp.float32),
            pltpu.VMEM((_ROWW,), jnp.float32),
            pltpu.VMEM((_CHUNK, 16), jnp.float32),
            pltpu.VMEM((_CHUNK, 16), jnp.float32),
            pltpu.SemaphoreType.DMA,
        ],
    )
    def k(table_hbm, z1_hbm, qidx_hbm, psum_hbm, head_hbm,
          idx_v, rows_v, z1_v, ps_v, hd_v, sem):
        wid = lax.axis_index("s") * 2 + lax.axis_index("c")
        g0 = wid * gpw + jnp.minimum(wid, rem)
        cnt = gpw + jnp.where(wid < rem, 1, 0)

        def group_body(gi, carry):
            g = g0 + gi
            b = g // _P
            zrow = b * _PPAD + (g - b * _P)
            pltpu.sync_copy(z1_hbm.at[pl.ds(zrow * _ROWW, _ROWW)], z1_v)
            zc = [z1_v[pl.ds(cc * 16, 16)] for cc in range(_C // 16)]
            for half in range(2):
                pbase = g * _NEG + half * _CHUNK
                pltpu.sync_copy(qidx_hbm.at[pl.ds(pbase, _CHUNK)], idx_v)
                pltpu.async_copy(table_hbm.at[idx_v], rows_v, sem).wait()

                def pair_body(j4, carry2):
                    for u in range(4):
                        j = j4 * 4 + u
                        acc = rows_v[j, pl.ds(0, 16)] * zc[0]
                        for cc in range(1, _C // 16):
                            acc = acc + rows_v[j, pl.ds(cc * 16, 16)] * zc[cc]
                        ps_v[j, pl.ds(0, 16)] = acc
                        hd_v[j, pl.ds(0, 16)] = rows_v[j, pl.ds(_C, 16)]
                    return carry2

                lax.fori_loop(0, _CHUNK // 4, pair_body, 0)
                pltpu.sync_copy(ps_v, psum_hbm.at[pl.ds(pbase, _CHUNK)])
                pltpu.sync_copy(hd_v, head_hbm.at[pl.ds(pbase, _CHUNK)])
            return carry

        lax.fori_loop(0, cnt, group_body, 0)

    return k(table, z1flat, qidx)


def _stage2_body(ps_ref, hd_ref, ew_ref, aux_ref, out_ref):
    b = pl.program_id(0)
    t = pl.program_id(1)
    ps = ps_ref[0]
    dot = jnp.sum(ps, axis=-1)
    hd = hd_ref[0]
    n2 = hd[:, :, 0]
    dr = hd[:, :, 1] - aux_ref[0, :, 1:2]
    dg = hd[:, :, 2] - aux_ref[0, :, 2:3]
    db = hd[:, :, 3] - aux_ref[0, :, 3:4]
    rgbsq = dr * dr + dg * dg + db * db
    w = ew_ref[0] + _RGBC * jnp.sqrt(rgbsq)
    n1 = aux_ref[0, :, 0:1]
    denom = jnp.maximum(n1 * n2, _EPS)
    val = jnp.minimum(jnp.abs(dot * w / denom), 1.0)
    part = jnp.sum(val, axis=0)[None, :]

    @pl.when(jnp.logical_and(b == 0, t == 0))
    def _():
        out_ref[...] = jnp.zeros_like(out_ref)

    out_ref[pl.ds(b, 1), :] += part


def _stage2(psums, heads, eucw, aux):
    return pl.pallas_call(
        _stage2_body,
        grid=(_B, _NG),
        in_specs=[
            pl.BlockSpec((1, _NG, _NEG, 16), lambda b, t: (b * _NG + t, 0, 0, 0)),
            pl.BlockSpec((1, _NG, _NEG, 16), lambda b, t: (b * _NG + t, 0, 0, 0)),
            pl.BlockSpec((1, _NG, _NEG), lambda b, t: (b * _NG + t, 0, 0)),
            pl.BlockSpec((1, _NG, 4), lambda b, t: (b * _NG + t, 0, 0)),
        ],
        out_specs=pl.BlockSpec((_B, _NEG), lambda b, t: (0, 0)),
        out_shape=jax.ShapeDtypeStruct((_B, _NEG), jnp.float32),
    )(psums, heads, eucw, aux)


def _final_body(s_ref, nsq_ref, out_ref):
    sneg = s_ref[...] * (1.0 / (_P * 2.0))
    nsq = nsq_ref[...]
    s0 = jnp.minimum(jnp.abs(nsq / jnp.maximum(nsq, _EPS)), 1.0)
    sim0 = jnp.sum(s0, axis=1) * (1.0 / _P)
    logp = jnp.clip(jnp.log(sim0), -100.0, None)
    log1m = jnp.clip(jnp.log(1.0 - sneg), -100.0, None)
    loss_b = -(logp + jnp.sum(log1m, axis=1)) * (1.0 / (_NEG + 1))
    loss = jnp.mean(loss_b)
    out2 = jnp.mean(sim0)
    out3 = jnp.sum(sneg) * (2.0 / (_NEG * _B))
    lane = lax.broadcasted_iota(jnp.int32, (1, 128), 1)
    out_ref[...] = (jnp.where(lane == 0, loss, 0.0)
                    + jnp.where(lane == 1, out2, 0.0)
                    + jnp.where(lane == 2, out3, 0.0))


def _final(S, nsq):
    return pl.pallas_call(
        _final_body,
        out_shape=jax.ShapeDtypeStruct((1, 128), jnp.float32),
    )(S, nsq)


def kernel(views_1, views_2, img):
    starts, qidx_np, eucw_np = _host_constants()
    qidx = jnp.asarray(qidx_np)
    eucw = jnp.asarray(eucw_np)

    v2 = views_2.reshape(_B, _C, _PIX)
    img0 = img[0].reshape(3, _PIX)
    table = _prep_table(v2, img0)

    z1list, rgblist = [], []
    for b, (si, sj) in enumerate(starts):
        z1list.append(
            lax.slice(views_1, (b, 0, si, sj), (b + 1, _C, si + 209, sj + 209),
                      (1, 1, _GS, _GS)).reshape(_C, _P))
        rgblist.append(
            lax.slice(img, (0, 0, si, sj), (1, 3, si + 209, sj + 209),
                      (1, 1, _GS, _GS)).reshape(3, _P))
    z1g = jnp.pad(jnp.stack(z1list), ((0, 0), (0, 0), (0, _PPAD - _P)))
    rgbg = jnp.pad(jnp.stack(rgblist), ((0, 0), (0, 0), (0, _PPAD - _P)))
    z1tab, nsq = _prep_z1(z1g, rgbg)

    psums, heads = _sc_main(table, z1tab.reshape(-1), qidx)

    n1 = z1tab[:, :_P, _C]
    posrgb = z1tab[:, :_P, _C + 1:_C + 4]
    aux = jnp.concatenate([n1[..., None], posrgb], -1).reshape(_GTILE, _NG, 4)

    S = _stage2(psums.reshape(_GTILE, _NG, _NEG, 16),
                heads.reshape(_GTILE, _NG, _NEG, 16), eucw, aux)
    out = _final(S, nsq)
    return out[0, 0], out[0, 1], out[0, 2]


# speedup vs baseline: 8.2227x; 1.1031x over previous
import functools

import numpy as np
import jax
import jax.numpy as jnp
from jax import lax
from jax.experimental import pallas as pl
from jax.experimental.pallas import tpu as pltpu
from jax.experimental.pallas import tpu_sc as plsc

_B, _C, _H, _W = 8, 96, 224, 224
_GS = 8
_NG = 27
_P = _NG * _NG
_NEG = 256
_PIX = _H * _W
_ROWW = 128
_PPAD = 768
_GROUPS = _B * _P
_GTILE = _GROUPS // _NG
_NPAIR = _GROUPS * _NEG
_NW = 32
_CHUNK = 128
_TP = 512
_RGBC = float(0.2 / np.sqrt(3.0))
_EPS = 1e-8


@functools.lru_cache(maxsize=1)
def _host_constants():
    rng = np.random.RandomState(0)
    starts = []
    qidx = np.empty((_B, _P, _NEG), np.int32)
    eucw = np.empty((_B, _P, _NEG), np.float32)
    max_euc = np.sqrt(float((_H - 1) ** 2 + (_W - 1) ** 2))
    base = np.arange(0, _H - _GS, _GS)
    for b in range(_B):
        si = int(rng.choice(_GS, 1)[0])
        sj = int(rng.choice(_GS, 1)[0])
        starts.append((si, sj))
        ic, jc = np.meshgrid(base, base, indexing="ij")
        ic = ic + si
        jc = jc + sj
        neg_i = rng.randint(0, _H, size=(_P, _NEG))
        neg_j = rng.randint(0, _W, size=(_P, _NEG))
        qidx[b] = (b * _PIX + neg_i * _W + neg_j).astype(np.int32)
        coords = np.stack([ic.reshape(_P), jc.reshape(_P)], 0).astype(np.float32)
        negs = np.stack([neg_i, neg_j], 0).astype(np.float32)
        euc = np.linalg.norm(coords[:, :, None] - negs, axis=0) / max_euc
        eucw[b] = (euc * 0.8).astype(np.float32)
    sel = np.zeros((7, 128, 128), np.float32)
    for gblk in range(8):
        s = 16 * gblk
        sel[0, s:s + 16, s:s + 16] = 1.0
        sel[1, s + 0, s:s + 16] = 1.0
        sel[2, s + 4, s:s + 16] = 1.0
        sel[3, s + 5, s:s + 16] = 1.0
        sel[4, s + 1, s:s + 16] = 1.0
        sel[4, s + 6, s:s + 16] = -1.0
        sel[5, s + 2, s:s + 16] = 1.0
        sel[5, s + 7, s:s + 16] = -1.0
        sel[6, s + 3, s:s + 16] = 1.0
        sel[6, s + 8, s:s + 16] = -1.0
    return starts, qidx.reshape(-1), eucw.reshape(-1), sel


def _prep_table_body(v2_ref, img_ref, out_ref):
    x = v2_ref[0]
    out_ref[0, :, 0:_C] = x.T
    nrm = jnp.sqrt(jnp.sum(x * x, axis=0))
    out_ref[0, :, _C:_C + 1] = nrm[:, None]
    out_ref[0, :, _C + 1:_C + 4] = img_ref[...].T
    out_ref[0, :, _C + 4:_ROWW] = jnp.zeros((_TP, _ROWW - _C - 4), jnp.float32)


def _prep_table(v2, img0):
    out = pl.pallas_call(
        _prep_table_body,
        grid=(_B, _PIX // _TP),
        in_specs=[
            pl.BlockSpec((1, _C, _TP), lambda b, t: (b, 0, t)),
            pl.BlockSpec((3, _TP), lambda b, t: (0, t)),
        ],
        out_specs=pl.BlockSpec((1, _TP, _ROWW), lambda b, t: (b, t, 0)),
        out_shape=jax.ShapeDtypeStruct((_B, _PIX, _ROWW), jnp.float32),
    )(v2, img0)
    return out.reshape(_B * _PIX, _ROWW)


def _prep_z1_body(z_ref, rgb_ref, out_ref, nsq_ref):
    for b in range(_B):
        x = z_ref[b]
        out_ref[b, :, 0:_C] = x.T
        nsq = jnp.sum(x * x, axis=0)
        out_ref[b, :, _C:_C + 1] = jnp.sqrt(nsq)[:, None]
        out_ref[b, :, _C + 1:_C + 4] = rgb_ref[b].T
        out_ref[b, :, _C + 4:_ROWW] = jnp.zeros((_PPAD, _ROWW - _C - 4), jnp.float32)
        nsq_ref[b:b + 1, :] = nsq[None, :]


def _prep_z1(z1g, rgbg):
    return pl.pallas_call(
        _prep_z1_body,
        out_shape=(
            jax.ShapeDtypeStruct((_B, _PPAD, _ROWW), jnp.float32),
            jax.ShapeDtypeStruct((_B, _PPAD), jnp.float32),
        ),
    )(z1g, rgbg)


def _sc_main(table, z1flat, qidx, eucw):
    gpw = _GROUPS // _NW
    rem = _GROUPS % _NW
    mesh = plsc.VectorSubcoreMesh(core_axis_name="c", subcore_axis_name="s")

    @functools.partial(
        pl.kernel,
        mesh=mesh,
        out_type=(
            jax.ShapeDtypeStruct((_NPAIR, 16), jnp.float32),
            jax.ShapeDtypeStruct((_NPAIR, 16), jnp.float32),
        ),
        scratch_types=[
            pltpu.VMEM((_CHUNK,), jnp.int32),
            pltpu.VMEM((_CHUNK, _ROWW), jnp.float32),
            pltpu.VMEM((_ROWW,), jnp.float32),
            pltpu.VMEM((_CHUNK,), jnp.float32),
            pltpu.VMEM((_CHUNK, 16), jnp.float32),
            pltpu.VMEM((_CHUNK, 16), jnp.float32),
            pltpu.SemaphoreType.DMA,
        ],
    )
    def k(table_hbm, z1_hbm, qidx_hbm, eucw_hbm, psum_hbm, head_hbm,
          idx_v, rows_v, z1_v, ew_v, ps_v, hd_v, sem):
        wid = lax.axis_index("s") * 2 + lax.axis_index("c")
        g0 = wid * gpw + jnp.minimum(wid, rem)
        cnt = gpw + jnp.where(wid < rem, 1, 0)
        lane = lax.iota(jnp.int32, 16)

        def group_body(gi, carry):
            g = g0 + gi
            b = g // _P
            zrow = b * _PPAD + (g - b * _P)
            pltpu.sync_copy(z1_hbm.at[pl.ds(zrow * _ROWW, _ROWW)], z1_v)
            zc = [z1_v[pl.ds(cc * 16, 16)] for cc in range(_C // 16)]
            zhead = z1_v[pl.ds(_C, 16)]
            zpart = jnp.zeros((16,), jnp.float32)
            for kk in range(4):
                bc = jnp.take(zhead, jnp.full((16,), kk, jnp.int32))
                zpart = zpart + jnp.where(lane == 5 + kk, bc, 0.0)
            for half in range(2):
                pbase = g * _NEG + half * _CHUNK
                pltpu.sync_copy(qidx_hbm.at[pl.ds(pbase, _CHUNK)], idx_v)
                pltpu.async_copy(table_hbm.at[idx_v], rows_v, sem).wait()
                pltpu.sync_copy(eucw_hbm.at[pl.ds(pbase, _CHUNK)], ew_v)

                def pair_body(blk, carry2):
                    evec = ew_v[pl.ds(blk * 16, 16)]
                    for u in range(16):
                        j = blk * 16 + u
                        acc = rows_v[j, pl.ds(0, 16)] * zc[0]
                        for cc in range(1, _C // 16):
                            acc = acc + rows_v[j, pl.ds(cc * 16, 16)] * zc[cc]
                        ps_v[j, pl.ds(0, 16)] = acc
                        head = rows_v[j, pl.ds(_C, 16)]
                        hd_v[j, pl.ds(0, 16)] = (
                            head + zpart + jnp.where(lane == 4, evec[u], 0.0))
                    return carry2

                lax.fori_loop(0, _CHUNK // 16, pair_body, 0)
                pltpu.sync_copy(ps_v, psum_hbm.at[pl.ds(pbase, _CHUNK)])
                pltpu.sync_copy(hd_v, head_hbm.at[pl.ds(pbase, _CHUNK)])
            return carry

        lax.fori_loop(0, cnt, group_body, 0)

    return k(table, z1flat, qidx, eucw)


_ROWS2 = _NG * _NEG * 16 // 128


def _stage2_body(ps_ref, hd_ref, sel_ref, out_ref):
    b = pl.program_id(0)
    t = pl.program_id(1)
    dn = (((1,), (0,)), ((), ()))
    X = ps_ref[0]
    H = hd_ref[0]

    def mm(A, k):
        return lax.dot_general(A, sel_ref[k], dn,
                               preferred_element_type=jnp.float32)

    dot = mm(X, 0)
    n2 = mm(H, 1)
    ew = mm(H, 2)
    n1 = mm(H, 3)
    dr = mm(H, 4)
    dg = mm(H, 5)
    db = mm(H, 6)
    rgbsq = dr * dr + dg * dg + db * db
    w = ew + _RGBC * jnp.sqrt(rgbsq)
    denom = jnp.maximum(n1 * n2, _EPS)
    val = jnp.minimum(jnp.abs(dot * w / denom), 1.0)
    part = jnp.sum(val.reshape(_NG, _ROWS2 // _NG, 128), axis=0)

    @pl.when(jnp.logical_and(b == 0, t == 0))
    def _():
        out_ref[...] = jnp.zeros_like(out_ref)

    out_ref[pl.ds(b, 1)] += part[None]


def _stage2(psums, heads, sel):
    return pl.pallas_call(
        _stage2_body,
        grid=(_B, _NG),
        in_specs=[
            pl.BlockSpec((1, _ROWS2, 128), lambda b, t: (b * _NG + t, 0, 0)),
            pl.BlockSpec((1, _ROWS2, 128), lambda b, t: (b * _NG + t, 0, 0)),
            pl.BlockSpec((7, 128, 128), lambda b, t: (0, 0, 0)),
        ],
        out_specs=pl.BlockSpec((_B, _ROWS2 // _NG, 128), lambda b, t: (0, 0, 0)),
        out_shape=jax.ShapeDtypeStruct((_B, _ROWS2 // _NG, 128), jnp.float32),
    )(psums, heads, sel)


def _final_body(s_ref, nsq_ref, out_ref):
    s4 = s_ref[...].reshape(_B, 32, 8, 16)
    lane0 = (lax.broadcasted_iota(jnp.int32, (16,), 0) == 0).astype(jnp.float32)
    S = lax.dot_general(s4, lane0, (((3,), (0,)), ((), ()))).reshape(_B, _NEG)
    sneg = S * (1.0 / (_P * 2.0))
    nsq = nsq_ref[...]
    s0 = jnp.minimum(jnp.abs(nsq / jnp.maximum(nsq, _EPS)), 1.0)
    sim0 = jnp.sum(s0, axis=1) * (1.0 / _P)
    logp = jnp.clip(jnp.log(sim0), -100.0, None)
    log1m = jnp.clip(jnp.log(1.0 - sneg), -100.0, None)
    loss_b = -(logp + jnp.sum(log1m, axis=1)) * (1.0 / (_NEG + 1))
    loss = jnp.mean(loss_b)
    out2 = jnp.mean(sim0)
    out3 = jnp.sum(sneg) * (2.0 / (_NEG * _B))
    lane = lax.broadcasted_iota(jnp.int32, (1, 128), 1)
    out_ref[...] = (jnp.where(lane == 0, loss, 0.0)
                    + jnp.where(lane == 1, out2, 0.0)
                    + jnp.where(lane == 2, out3, 0.0))


def _final(S, nsq):
    return pl.pallas_call(
        _final_body,
        out_shape=jax.ShapeDtypeStruct((1, 128), jnp.float32),
    )(S, nsq)


def kernel(views_1, views_2, img):
    starts, qidx_np, eucw_np, sel_np = _host_constants()
    qidx = jnp.asarray(qidx_np)
    eucw = jnp.asarray(eucw_np)
    sel = jnp.asarray(sel_np)

    v2 = views_2.reshape(_B, _C, _PIX)
    img0 = img[0].reshape(3, _PIX)
    table = _prep_table(v2, img0)

    z1list, rgblist = [], []
    for b, (si, sj) in enumerate(starts):
        z1list.append(
            lax.slice(views_1, (b, 0, si, sj), (b + 1, _C, si + 209, sj + 209),
                      (1, 1, _GS, _GS)).reshape(_C, _P))
        rgblist.append(
            lax.slice(img, (0, 0, si, sj), (1, 3, si + 209, sj + 209),
                      (1, 1, _GS, _GS)).reshape(3, _P))
    z1g = jnp.pad(jnp.stack(z1list), ((0, 0), (0, 0), (0, _PPAD - _P)))
    rgbg = jnp.pad(jnp.stack(rgblist), ((0, 0), (0, 0), (0, _PPAD - _P)))
    z1tab, nsq = _prep_z1(z1g, rgbg)

    psums, heads = _sc_main(table, z1tab.reshape(-1), qidx, eucw)

    S = _stage2(psums.reshape(_GTILE, _ROWS2, 128),
                heads.reshape(_GTILE, _ROWS2, 128), sel)
    out = _final(S, nsq)
    return out[0, 0], out[0, 1], out[0, 2]


# speedup vs baseline: 10.4363x; 1.2692x over previous
import functools

import numpy as np
import jax
import jax.numpy as jnp
from jax import lax
from jax.experimental import pallas as pl
from jax.experimental.pallas import tpu as pltpu
from jax.experimental.pallas import tpu_sc as plsc

_B, _C, _H, _W = 8, 96, 224, 224
_GS = 8
_NG = 27
_P = _NG * _NG
_NEG = 256
_PIX = _H * _W
_ROWW = 128
_PPAD = 768
_GROUPS = _B * _P
_GTILE = _GROUPS // _NG
_NPAIR = _GROUPS * _NEG
_NW = 32
_CHUNK = 128
_TP = 512
_RGBC = float(0.2 / np.sqrt(3.0))
_EPS = 1e-8


@functools.lru_cache(maxsize=1)
def _host_constants():
    rng = np.random.RandomState(0)
    starts = []
    qidx = np.empty((_B, _P, _NEG), np.int32)
    eucw = np.empty((_B, _P, _NEG), np.float32)
    max_euc = np.sqrt(float((_H - 1) ** 2 + (_W - 1) ** 2))
    base = np.arange(0, _H - _GS, _GS)
    for b in range(_B):
        si = int(rng.choice(_GS, 1)[0])
        sj = int(rng.choice(_GS, 1)[0])
        starts.append((si, sj))
        ic, jc = np.meshgrid(base, base, indexing="ij")
        ic = ic + si
        jc = jc + sj
        neg_i = rng.randint(0, _H, size=(_P, _NEG))
        neg_j = rng.randint(0, _W, size=(_P, _NEG))
        qidx[b] = (b * _PIX + neg_i * _W + neg_j).astype(np.int32)
        coords = np.stack([ic.reshape(_P), jc.reshape(_P)], 0).astype(np.float32)
        negs = np.stack([neg_i, neg_j], 0).astype(np.float32)
        euc = np.linalg.norm(coords[:, :, None] - negs, axis=0) / max_euc
        eucw[b] = (euc * 0.8).astype(np.float32)
    sel = np.zeros((7, 128, 128), np.float32)
    for gblk in range(8):
        s = 16 * gblk
        sel[0, s:s + 16, s:s + 16] = 1.0
        sel[1, s + 0, s:s + 16] = 1.0
        sel[2, s + 4, s:s + 16] = 1.0
        sel[3, s + 5, s:s + 16] = 1.0
        sel[4, s + 1, s:s + 16] = 1.0
        sel[4, s + 6, s:s + 16] = -1.0
        sel[5, s + 2, s:s + 16] = 1.0
        sel[5, s + 7, s:s + 16] = -1.0
        sel[6, s + 3, s:s + 16] = 1.0
        sel[6, s + 8, s:s + 16] = -1.0
    return starts, qidx.reshape(-1), eucw.reshape(-1), sel


def _prep_table_body(v2_ref, img_ref, out_ref):
    x = v2_ref[0]
    out_ref[0, :, 0:_C] = x.T
    nrm = jnp.sqrt(jnp.sum(x * x, axis=0))
    out_ref[0, :, _C:_C + 1] = nrm[:, None]
    out_ref[0, :, _C + 1:_C + 4] = img_ref[...].T
    out_ref[0, :, _C + 4:_ROWW] = jnp.zeros((_TP, _ROWW - _C - 4), jnp.float32)


def _prep_table(v2, img0):
    out = pl.pallas_call(
        _prep_table_body,
        grid=(_B, _PIX // _TP),
        in_specs=[
            pl.BlockSpec((1, _C, _TP), lambda b, t: (b, 0, t)),
            pl.BlockSpec((3, _TP), lambda b, t: (0, t)),
        ],
        out_specs=pl.BlockSpec((1, _TP, _ROWW), lambda b, t: (b, t, 0)),
        out_shape=jax.ShapeDtypeStruct((_B, _PIX, _ROWW), jnp.float32),
    )(v2, img0)
    return out.reshape(_B * _PIX, _ROWW)


def _prep_z1_body(z_ref, rgb_ref, out_ref, nsq_ref):
    for b in range(_B):
        x = z_ref[b]
        out_ref[b, :, 0:_C] = x.T
        nsq = jnp.sum(x * x, axis=0)
        out_ref[b, :, _C:_C + 1] = jnp.sqrt(nsq)[:, None]
        out_ref[b, :, _C + 1:_C + 4] = rgb_ref[b].T
        out_ref[b, :, _C + 4:_ROWW] = jnp.zeros((_PPAD, _ROWW - _C - 4), jnp.float32)
        nsq_ref[b:b + 1, :] = nsq[None, :]


def _prep_z1(z1g, rgbg):
    return pl.pallas_call(
        _prep_z1_body,
        out_shape=(
            jax.ShapeDtypeStruct((_B, _PPAD, _ROWW), jnp.float32),
            jax.ShapeDtypeStruct((_B, _PPAD), jnp.float32),
        ),
    )(z1g, rgbg)


def _sc_main(table, z1flat, qidx, eucw):
    gpw = _GROUPS // _NW
    rem = _GROUPS % _NW
    mesh = plsc.VectorSubcoreMesh(core_axis_name="c", subcore_axis_name="s")

    @functools.partial(
        pl.kernel,
        mesh=mesh,
        out_type=(
            jax.ShapeDtypeStruct((_NPAIR * 16,), jnp.float32),
            jax.ShapeDtypeStruct((_NPAIR * 16,), jnp.float32),
        ),
        scratch_types=[
            pltpu.VMEM((_CHUNK,), jnp.int32),
            pltpu.VMEM((_CHUNK, _ROWW), jnp.float32),
            pltpu.VMEM((_ROWW,), jnp.float32),
            pltpu.VMEM((_CHUNK,), jnp.float32),
            pltpu.VMEM((_CHUNK * 16,), jnp.float32),
            pltpu.VMEM((_CHUNK * 16,), jnp.float32),
            pltpu.SemaphoreType.DMA,
        ],
    )
    def k(table_hbm, z1_hbm, qidx_hbm, eucw_hbm, psum_hbm, head_hbm,
          idx_v, rows_v, z1_v, ew_v, ps_v, hd_v, sem):
        wid = lax.axis_index("s") * 2 + lax.axis_index("c")
        g0 = wid * gpw + jnp.minimum(wid, rem)
        cnt = gpw + jnp.where(wid < rem, 1, 0)
        lane = lax.iota(jnp.int32, 16)

        def group_body(gi, carry):
            g = g0 + gi
            b = g // _P
            zrow = b * _PPAD + (g - b * _P)
            pltpu.sync_copy(z1_hbm.at[pl.ds(zrow * _ROWW, _ROWW)], z1_v)
            zc = [z1_v[pl.ds(cc * 16, 16)] for cc in range(_C // 16)]
            zhead = z1_v[pl.ds(_C, 16)]
            zpart = jnp.zeros((16,), jnp.float32)
            for kk in range(4):
                bc = jnp.take(zhead, jnp.full((16,), kk, jnp.int32))
                zpart = zpart + jnp.where(lane == 5 + kk, bc, 0.0)
            for half in range(2):
                pbase = g * _NEG + half * _CHUNK
                pltpu.sync_copy(qidx_hbm.at[pl.ds(pbase, _CHUNK)], idx_v)
                pltpu.async_copy(table_hbm.at[idx_v], rows_v, sem).wait()
                pltpu.sync_copy(eucw_hbm.at[pl.ds(pbase, _CHUNK)], ew_v)

                def pair_body(blk, carry2):
                    evec = ew_v[pl.ds(blk * 16, 16)]
                    for u in range(16):
                        j = blk * 16 + u
                        acc = rows_v[j, pl.ds(0, 16)] * zc[0]
                        for cc in range(1, _C // 16):
                            acc = acc + rows_v[j, pl.ds(cc * 16, 16)] * zc[cc]
                        ps_v[pl.ds(j * 16, 16)] = acc
                        head = rows_v[j, pl.ds(_C, 16)]
                        hd_v[pl.ds(j * 16, 16)] = (
                            head + zpart + jnp.where(lane == 4, evec[u], 0.0))
                    return carry2

                lax.fori_loop(0, _CHUNK // 16, pair_body, 0)
                pltpu.sync_copy(ps_v, psum_hbm.at[pl.ds(pbase * 16, _CHUNK * 16)])
                pltpu.sync_copy(hd_v, head_hbm.at[pl.ds(pbase * 16, _CHUNK * 16)])
            return carry

        lax.fori_loop(0, cnt, group_body, 0)

    return k(table, z1flat, qidx, eucw)


_ROWS2 = _NG * _NEG * 16 // 128


def _stage2_body(ps_ref, hd_ref, sel_ref, out_ref):
    b = pl.program_id(0)
    t = pl.program_id(1)
    dn = (((1,), (0,)), ((), ()))
    X = ps_ref[0]
    H = hd_ref[0]

    def mm(A, k):
        return lax.dot_general(A, sel_ref[k], dn,
                               preferred_element_type=jnp.float32)

    dot = mm(X, 0)
    n2 = mm(H, 1)
    ew = mm(H, 2)
    n1 = mm(H, 3)
    dr = mm(H, 4)
    dg = mm(H, 5)
    db = mm(H, 6)
    rgbsq = dr * dr + dg * dg + db * db
    w = ew + _RGBC * jnp.sqrt(rgbsq)
    denom = jnp.maximum(n1 * n2, _EPS)
    val = jnp.minimum(jnp.abs(dot * w / denom), 1.0)
    part = jnp.sum(val.reshape(_NG, _ROWS2 // _NG, 128), axis=0)

    @pl.when(jnp.logical_and(b == 0, t == 0))
    def _():
        out_ref[...] = jnp.zeros_like(out_ref)

    out_ref[pl.ds(b, 1)] += part[None]


def _stage2(psums, heads, sel):
    return pl.pallas_call(
        _stage2_body,
        grid=(_B, _NG),
        in_specs=[
            pl.BlockSpec((1, _ROWS2, 128), lambda b, t: (b * _NG + t, 0, 0)),
            pl.BlockSpec((1, _ROWS2, 128), lambda b, t: (b * _NG + t, 0, 0)),
            pl.BlockSpec((7, 128, 128), lambda b, t: (0, 0, 0)),
        ],
        out_specs=pl.BlockSpec((_B, _ROWS2 // _NG, 128), lambda b, t: (0, 0, 0)),
        out_shape=jax.ShapeDtypeStruct((_B, _ROWS2 // _NG, 128), jnp.float32),
    )(psums, heads, sel)


def _final_body(s_ref, nsq_ref, out_ref):
    s4 = s_ref[...].reshape(_B, 32, 8, 16)
    lane0 = (lax.broadcasted_iota(jnp.int32, (16,), 0) == 0).astype(jnp.float32)
    S = lax.dot_general(s4, lane0, (((3,), (0,)), ((), ()))).reshape(_B, _NEG)
    sneg = S * (1.0 / (_P * 2.0))
    nsq = nsq_ref[...]
    s0 = jnp.minimum(jnp.abs(nsq / jnp.maximum(nsq, _EPS)), 1.0)
    sim0 = jnp.sum(s0, axis=1) * (1.0 / _P)
    logp = jnp.clip(jnp.log(sim0), -100.0, None)
    log1m = jnp.clip(jnp.log(1.0 - sneg), -100.0, None)
    loss_b = -(logp + jnp.sum(log1m, axis=1)) * (1.0 / (_NEG + 1))
    loss = jnp.mean(loss_b)
    out2 = jnp.mean(sim0)
    out3 = jnp.sum(sneg) * (2.0 / (_NEG * _B))
    lane = lax.broadcasted_iota(jnp.int32, (1, 128), 1)
    out_ref[...] = (jnp.where(lane == 0, loss, 0.0)
                    + jnp.where(lane == 1, out2, 0.0)
                    + jnp.where(lane == 2, out3, 0.0))


def _final(S, nsq):
    return pl.pallas_call(
        _final_body,
        out_shape=jax.ShapeDtypeStruct((1, 128), jnp.float32),
    )(S, nsq)


def kernel(views_1, views_2, img):
    starts, qidx_np, eucw_np, sel_np = _host_constants()
    qidx = jnp.asarray(qidx_np)
    eucw = jnp.asarray(eucw_np)
    sel = jnp.asarray(sel_np)

    v2 = views_2.reshape(_B, _C, _PIX)
    img0 = img[0].reshape(3, _PIX)
    table = _prep_table(v2, img0)

    z1list, rgblist = [], []
    for b, (si, sj) in enumerate(starts):
        z1list.append(
            lax.slice(views_1, (b, 0, si, sj), (b + 1, _C, si + 209, sj + 209),
                      (1, 1, _GS, _GS)).reshape(_C, _P))
        rgblist.append(
            lax.slice(img, (0, 0, si, sj), (1, 3, si + 209, sj + 209),
                      (1, 1, _GS, _GS)).reshape(3, _P))
    z1g = jnp.pad(jnp.stack(z1list), ((0, 0), (0, 0), (0, _PPAD - _P)))
    rgbg = jnp.pad(jnp.stack(rgblist), ((0, 0), (0, 0), (0, _PPAD - _P)))
    z1tab, nsq = _prep_z1(z1g, rgbg)

    psums, heads = _sc_main(table, z1tab.reshape(-1), qidx, eucw)

    S = _stage2(psums.reshape(_GTILE, _ROWS2, 128),
                heads.reshape(_GTILE, _ROWS2, 128), sel)
    out = _final(S, nsq)
    return out[0, 0], out[0, 1], out[0, 2]


# speedup vs baseline: 14.4102x; 1.3808x over previous
import functools

import numpy as np
import jax
import jax.numpy as jnp
from jax import lax
from jax.experimental import pallas as pl
from jax.experimental.pallas import tpu as pltpu
from jax.experimental.pallas import tpu_sc as plsc

_B, _C, _H, _W = 8, 96, 224, 224
_GS = 8
_NG = 27
_P = _NG * _NG
_NEG = 256
_PIX = _H * _W
_ROWW = 128
_PPAD = 768
_GROUPS = _B * _P
_GTILE = _GROUPS // _NG
_NPAIR = _GROUPS * _NEG
_NW = 32
_SB = 8
_GPAD = 5840
_CHUNK = 128
_TP = 512
_RGBC = float(0.2 / np.sqrt(3.0))
_EPS = 1e-8


@functools.lru_cache(maxsize=1)
def _host_constants():
    rng = np.random.RandomState(0)
    starts = []
    qidx = np.empty((_B, _P, _NEG), np.int32)
    eucw = np.empty((_B, _P, _NEG), np.float32)
    max_euc = np.sqrt(float((_H - 1) ** 2 + (_W - 1) ** 2))
    base = np.arange(0, _H - _GS, _GS)
    for b in range(_B):
        si = int(rng.choice(_GS, 1)[0])
        sj = int(rng.choice(_GS, 1)[0])
        starts.append((si, sj))
        ic, jc = np.meshgrid(base, base, indexing="ij")
        ic = ic + si
        jc = jc + sj
        neg_i = rng.randint(0, _H, size=(_P, _NEG))
        neg_j = rng.randint(0, _W, size=(_P, _NEG))
        qidx[b] = (b * _PIX + neg_i * _W + neg_j).astype(np.int32)
        coords = np.stack([ic.reshape(_P), jc.reshape(_P)], 0).astype(np.float32)
        negs = np.stack([neg_i, neg_j], 0).astype(np.float32)
        euc = np.linalg.norm(coords[:, :, None] - negs, axis=0) / max_euc
        eucw[b] = (euc * 0.8).astype(np.float32)
    qidx = qidx.reshape(_GROUPS * 2, _CHUNK)
    qidx = np.concatenate(
        [qidx, np.zeros(((_GPAD - _GROUPS) * 2, _CHUNK), np.int32)], 0)
    eucw2 = np.zeros((_GPAD * _NEG,), np.float32)
    eucw2[:_GROUPS * _NEG] = eucw.reshape(-1)
    sel = np.zeros((7, 128, 128), np.float32)
    for gblk in range(8):
        s = 16 * gblk
        sel[0, s:s + 16, s:s + 16] = 1.0
        sel[1, s + 0, s:s + 16] = 1.0
        sel[2, s + 4, s:s + 16] = 1.0
        sel[3, s + 5, s:s + 16] = 1.0
        sel[4, s + 1, s:s + 16] = 1.0
        sel[4, s + 6, s:s + 16] = -1.0
        sel[5, s + 2, s:s + 16] = 1.0
        sel[5, s + 7, s:s + 16] = -1.0
        sel[6, s + 3, s:s + 16] = 1.0
        sel[6, s + 8, s:s + 16] = -1.0
    return starts, qidx, eucw2, sel


def _prep_table_body(v2_ref, img_ref, out_ref):
    x = v2_ref[0]
    out_ref[0, :, 0:_C] = x.T
    nrm = jnp.sqrt(jnp.sum(x * x, axis=0))
    out_ref[0, :, _C:_C + 1] = nrm[:, None]
    out_ref[0, :, _C + 1:_C + 4] = img_ref[...].T
    out_ref[0, :, _C + 4:_ROWW] = jnp.zeros((_TP, _ROWW - _C - 4), jnp.float32)


def _prep_table(v2, img0):
    out = pl.pallas_call(
        _prep_table_body,
        grid=(_B, _PIX // _TP),
        in_specs=[
            pl.BlockSpec((1, _C, _TP), lambda b, t: (b, 0, t)),
            pl.BlockSpec((3, _TP), lambda b, t: (0, t)),
        ],
        out_specs=pl.BlockSpec((1, _TP, _ROWW), lambda b, t: (b, t, 0)),
        out_shape=jax.ShapeDtypeStruct((_B, _PIX, _ROWW), jnp.float32),
    )(v2, img0)
    return out.reshape(_B * _PIX, _ROWW)


def _prep_z1_body(z_ref, rgb_ref, out_ref, nsq_ref):
    for b in range(_B):
        x = z_ref[b]
        out_ref[b, :, 0:_C] = x.T
        nsq = jnp.sum(x * x, axis=0)
        out_ref[b, :, _C:_C + 1] = jnp.sqrt(nsq)[:, None]
        out_ref[b, :, _C + 1:_C + 4] = rgb_ref[b].T
        out_ref[b, :, _C + 4:_ROWW] = jnp.zeros((_PPAD, _ROWW - _C - 4), jnp.float32)
        nsq_ref[b:b + 1, :] = nsq[None, :]


def _prep_z1(z1g, rgbg):
    return pl.pallas_call(
        _prep_z1_body,
        out_shape=(
            jax.ShapeDtypeStruct((_B, _PPAD, _ROWW), jnp.float32),
            jax.ShapeDtypeStruct((_B, _PPAD), jnp.float32),
        ),
    )(z1g, rgbg)


def _sc_main(table, z1flat, qidx, eucw):
    sbtot = _GROUPS // _SB
    sbbase = sbtot // _NW
    sbrem = sbtot % _NW
    mesh = plsc.VectorSubcoreMesh(core_axis_name="c", subcore_axis_name="s")

    @functools.partial(
        pl.kernel,
        mesh=mesh,
        out_type=(
            jax.ShapeDtypeStruct((_NPAIR * 16,), jnp.float32),
            jax.ShapeDtypeStruct((_NPAIR * 16,), jnp.float32),
        ),
        scratch_types=[
            pltpu.VMEM((2 * _SB, _CHUNK), jnp.int32),
            pltpu.VMEM((_CHUNK, _ROWW), jnp.float32),
            pltpu.VMEM((_CHUNK, _ROWW), jnp.float32),
            pltpu.VMEM((_SB * _ROWW,), jnp.float32),
            pltpu.VMEM((_SB * _NEG,), jnp.float32),
            pltpu.VMEM((_CHUNK * 16,), jnp.float32),
            pltpu.VMEM((_CHUNK * 16,), jnp.float32),
            pltpu.SemaphoreType.DMA,
            pltpu.SemaphoreType.DMA,
        ],
    )
    def k(table_hbm, z1_hbm, qidx_hbm, eucw_hbm, psum_hbm, head_hbm,
          idx_v, rows0_v, rows1_v, z1_v, ew_v, ps_v, hd_v, sem0, sem1):
        wid = lax.axis_index("s") * 2 + lax.axis_index("c")
        sb0 = wid * sbbase + jnp.minimum(wid, sbrem)
        nsb = sbbase + jnp.where(wid < sbrem, 1, 0)
        lane = lax.iota(jnp.int32, 16)

        def compute_chunk(rows_v, pbase, zc, zpart, ewbase):
            def pair_body(blk, carry2):
                evec = ew_v[pl.ds(ewbase + blk * 16, 16)]
                for u in range(16):
                    j = blk * 16 + u
                    acc = rows_v[j, pl.ds(0, 16)] * zc[0]
                    for cc in range(1, _C // 16):
                        acc = acc + rows_v[j, pl.ds(cc * 16, 16)] * zc[cc]
                    ps_v[pl.ds(j * 16, 16)] = acc
                    head = rows_v[j, pl.ds(_C, 16)]
                    hd_v[pl.ds(j * 16, 16)] = (
                        head + zpart + jnp.where(lane == 4, evec[u], 0.0))
                return carry2

            lax.fori_loop(0, _CHUNK // 16, pair_body, 0)
            pltpu.sync_copy(ps_v, psum_hbm.at[pl.ds(pbase * 16, _CHUNK * 16)])
            pltpu.sync_copy(hd_v, head_hbm.at[pl.ds(pbase * 16, _CHUNK * 16)])

        def sb_body(sb, carry):
            gbase = (sb0 + sb) * _SB
            pltpu.sync_copy(qidx_hbm.at[pl.ds(gbase * 2, 2 * _SB)], idx_v)
            pltpu.sync_copy(z1_hbm.at[pl.ds(gbase * _ROWW, _SB * _ROWW)], z1_v)
            pltpu.sync_copy(eucw_hbm.at[pl.ds(gbase * _NEG, _SB * _NEG)], ew_v)
            pltpu.async_copy(table_hbm.at[idx_v.at[0]], rows0_v, sem0)

            def g_body(s, carry2):
                g = gbase + s
                zoff = s * _ROWW
                zc = [z1_v[pl.ds(zoff + cc * 16, 16)] for cc in range(_C // 16)]
                zhead = z1_v[pl.ds(zoff + _C, 16)]
                zpart = jnp.zeros((16,), jnp.float32)
                for kk in range(4):
                    bc = jnp.take(zhead, jnp.full((16,), kk, jnp.int32))
                    zpart = zpart + jnp.where(lane == 5 + kk, bc, 0.0)
                pltpu.async_copy(table_hbm.at[idx_v.at[2 * s + 1]], rows1_v, sem1)
                pltpu.make_async_copy(
                    table_hbm.at[idx_v.at[0]], rows0_v, sem0).wait()
                compute_chunk(rows0_v, g * _NEG, zc, zpart, s * _NEG)

                @pl.when(s + 1 < _SB)
                def _():
                    pltpu.async_copy(
                        table_hbm.at[idx_v.at[2 * s + 2]], rows0_v, sem0)

                pltpu.make_async_copy(
                    table_hbm.at[idx_v.at[0]], rows1_v, sem1).wait()
                compute_chunk(rows1_v, g * _NEG + _CHUNK, zc, zpart,
                              s * _NEG + _CHUNK)
                return carry2

            lax.fori_loop(0, _SB, g_body, 0)
            return carry

        lax.fori_loop(0, nsb, sb_body, 0)

    return k(table, z1flat, qidx, eucw)


_ROWS2 = _NG * _NEG * 16 // 128


def _stage2_body(ps_ref, hd_ref, sel_ref, out_ref):
    b = pl.program_id(0)
    t = pl.program_id(1)
    dn = (((1,), (0,)), ((), ()))
    X = ps_ref[0]
    H = hd_ref[0]

    def mm(A, k):
        return lax.dot_general(A, sel_ref[k], dn,
                               preferred_element_type=jnp.float32)

    dot = mm(X, 0)
    n2 = mm(H, 1)
    ew = mm(H, 2)
    n1 = mm(H, 3)
    dr = mm(H, 4)
    dg = mm(H, 5)
    db = mm(H, 6)
    rgbsq = dr * dr + dg * dg + db * db
    w = ew + _RGBC * jnp.sqrt(rgbsq)
    denom = jnp.maximum(n1 * n2, _EPS)
    val = jnp.minimum(jnp.abs(dot * w / denom), 1.0)
    part = jnp.sum(val.reshape(_NG, _ROWS2 // _NG, 128), axis=0)

    @pl.when(jnp.logical_and(b == 0, t == 0))
    def _():
        out_ref[...] = jnp.zeros_like(out_ref)

    out_ref[pl.ds(b, 1)] += part[None]


def _stage2(psums, heads, sel):
    return pl.pallas_call(
        _stage2_body,
        grid=(_B, _NG),
        in_specs=[
            pl.BlockSpec((1, _ROWS2, 128), lambda b, t: (b * _NG + t, 0, 0)),
            pl.BlockSpec((1, _ROWS2, 128), lambda b, t: (b * _NG + t, 0, 0)),
            pl.BlockSpec((7, 128, 128), lambda b, t: (0, 0, 0)),
        ],
        out_specs=pl.BlockSpec((_B, _ROWS2 // _NG, 128), lambda b, t: (0, 0, 0)),
        out_shape=jax.ShapeDtypeStruct((_B, _ROWS2 // _NG, 128), jnp.float32),
    )(psums, heads, sel)


def _final_body(s_ref, nsq_ref, out_ref):
    s4 = s_ref[...].reshape(_B, 32, 8, 16)
    lane0 = (lax.broadcasted_iota(jnp.int32, (16,), 0) == 0).astype(jnp.float32)
    S = lax.dot_general(s4, lane0, (((3,), (0,)), ((), ()))).reshape(_B, _NEG)
    sneg = S * (1.0 / (_P * 2.0))
    nsq = nsq_ref[...]
    s0 = jnp.minimum(jnp.abs(nsq / jnp.maximum(nsq, _EPS)), 1.0)
    sim0 = jnp.sum(s0, axis=1) * (1.0 / _P)
    logp = jnp.clip(jnp.log(sim0), -100.0, None)
    log1m = jnp.clip(jnp.log(1.0 - sneg), -100.0, None)
    loss_b = -(logp + jnp.sum(log1m, axis=1)) * (1.0 / (_NEG + 1))
    loss = jnp.mean(loss_b)
    out2 = jnp.mean(sim0)
    out3 = jnp.sum(sneg) * (2.0 / (_NEG * _B))
    lane = lax.broadcasted_iota(jnp.int32, (1, 128), 1)
    out_ref[...] = (jnp.where(lane == 0, loss, 0.0)
                    + jnp.where(lane == 1, out2, 0.0)
                    + jnp.where(lane == 2, out3, 0.0))


def _final(S, nsq):
    return pl.pallas_call(
        _final_body,
        out_shape=jax.ShapeDtypeStruct((1, 128), jnp.float32),
    )(S, nsq)


def kernel(views_1, views_2, img):
    starts, qidx_np, eucw_np, sel_np = _host_constants()
    qidx = jnp.asarray(qidx_np)
    eucw = jnp.asarray(eucw_np)
    sel = jnp.asarray(sel_np)

    v2 = views_2.reshape(_B, _C, _PIX)
    img0 = img[0].reshape(3, _PIX)
    table = _prep_table(v2, img0)

    z1list, rgblist = [], []
    for b, (si, sj) in enumerate(starts):
        z1list.append(
            lax.slice(views_1, (b, 0, si, sj), (b + 1, _C, si + 209, sj + 209),
                      (1, 1, _GS, _GS)).reshape(_C, _P))
        rgblist.append(
            lax.slice(img, (0, 0, si, sj), (1, 3, si + 209, sj + 209),
                      (1, 1, _GS, _GS)).reshape(3, _P))
    z1g = jnp.pad(jnp.stack(z1list), ((0, 0), (0, 0), (0, _PPAD - _P)))
    rgbg = jnp.pad(jnp.stack(rgblist), ((0, 0), (0, 0), (0, _PPAD - _P)))
    z1tab, nsq = _prep_z1(z1g, rgbg)

    z1c = z1tab[:, :_P, :].reshape(_GROUPS, _ROWW)
    z1c = jnp.pad(z1c, ((0, _GPAD - _GROUPS), (0, 0)))
    psums, heads = _sc_main(table, z1c.reshape(-1), qidx, eucw)

    S = _stage2(psums.reshape(_GTILE, _ROWS2, 128),
                heads.reshape(_GTILE, _ROWS2, 128), sel)
    out = _final(S, nsq)
    return out[0, 0], out[0, 1], out[0, 2]


# speedup vs baseline: 18.9028x; 1.3118x over previous
import functools

import numpy as np
import jax
import jax.numpy as jnp
from jax import lax
from jax.experimental import pallas as pl
from jax.experimental.pallas import tpu as pltpu
from jax.experimental.pallas import tpu_sc as plsc

_B, _C, _H, _W = 8, 96, 224, 224
_GS = 8
_NG = 27
_P = _NG * _NG
_NEG = 256
_PIX = _H * _W
_ROWW = 128
_PPAD = 768
_GROUPS = _B * _P
_GTILE = _GROUPS // _NG
_NPAIR = _GROUPS * _NEG
_NW = 32
_SB = 8
_GPAD = 5840
_CHUNK = 128
_TP = 512
_RGBC = float(0.2 / np.sqrt(3.0))
_EPS = 1e-8


@functools.lru_cache(maxsize=1)
def _host_constants():
    rng = np.random.RandomState(0)
    starts = []
    qidx = np.empty((_B, _P, _NEG), np.int32)
    eucw = np.empty((_B, _P, _NEG), np.float32)
    max_euc = np.sqrt(float((_H - 1) ** 2 + (_W - 1) ** 2))
    base = np.arange(0, _H - _GS, _GS)
    for b in range(_B):
        si = int(rng.choice(_GS, 1)[0])
        sj = int(rng.choice(_GS, 1)[0])
        starts.append((si, sj))
        ic, jc = np.meshgrid(base, base, indexing="ij")
        ic = ic + si
        jc = jc + sj
        neg_i = rng.randint(0, _H, size=(_P, _NEG))
        neg_j = rng.randint(0, _W, size=(_P, _NEG))
        qidx[b] = (b * _PIX + neg_i * _W + neg_j).astype(np.int32)
        coords = np.stack([ic.reshape(_P), jc.reshape(_P)], 0).astype(np.float32)
        negs = np.stack([neg_i, neg_j], 0).astype(np.float32)
        euc = np.linalg.norm(coords[:, :, None] - negs, axis=0) / max_euc
        eucw[b] = (euc * 0.8).astype(np.float32)
    qidx = qidx.reshape(_GROUPS * 2, _CHUNK)
    qidx = np.concatenate(
        [qidx, np.zeros(((_GPAD - _GROUPS) * 2, _CHUNK), np.int32)], 0)
    eucw2 = np.zeros((_GPAD * _NEG,), np.float32)
    eucw2[:_GROUPS * _NEG] = eucw.reshape(-1)
    sel = np.zeros((7, 128, 128), np.float32)
    for gblk in range(8):
        s = 16 * gblk
        sel[0, s:s + 16, s:s + 16] = 1.0
        sel[1, s + 0, s:s + 16] = 1.0
        sel[2, s + 4, s:s + 16] = 1.0
        sel[3, s + 5, s:s + 16] = 1.0
        sel[4, s + 1, s:s + 16] = 1.0
        sel[4, s + 6, s:s + 16] = -1.0
        sel[5, s + 2, s:s + 16] = 1.0
        sel[5, s + 7, s:s + 16] = -1.0
        sel[6, s + 3, s:s + 16] = 1.0
        sel[6, s + 8, s:s + 16] = -1.0
    return starts, qidx, eucw2, sel


def _prep_table_body(v2_ref, img_ref, out_ref):
    x = v2_ref[0]
    out_ref[0, :, 0:_C] = x.T
    nrm = jnp.sqrt(jnp.sum(x * x, axis=0))
    out_ref[0, :, _C:_C + 1] = nrm[:, None]
    out_ref[0, :, _C + 1:_C + 4] = img_ref[...].T
    out_ref[0, :, _C + 4:_ROWW] = jnp.zeros((_TP, _ROWW - _C - 4), jnp.float32)


def _prep_table(v2, img0):
    out = pl.pallas_call(
        _prep_table_body,
        grid=(_B, _PIX // _TP),
        in_specs=[
            pl.BlockSpec((1, _C, _TP), lambda b, t: (b, 0, t)),
            pl.BlockSpec((3, _TP), lambda b, t: (0, t)),
        ],
        out_specs=pl.BlockSpec((1, _TP, _ROWW), lambda b, t: (b, t, 0)),
        out_shape=jax.ShapeDtypeStruct((_B, _PIX, _ROWW), jnp.float32),
    )(v2, img0)
    return out.reshape(_B * _PIX, _ROWW)


def _prep_z1_body(z_ref, rgb_ref, out_ref, nsq_ref):
    for b in range(_B):
        x = z_ref[b]
        out_ref[b, :, 0:_C] = x.T
        nsq = jnp.sum(x * x, axis=0)
        out_ref[b, :, _C:_C + 1] = jnp.sqrt(nsq)[:, None]
        out_ref[b, :, _C + 1:_C + 4] = rgb_ref[b].T
        out_ref[b, :, _C + 4:_ROWW] = jnp.zeros((_PPAD, _ROWW - _C - 4), jnp.float32)
        nsq_ref[b:b + 1, :] = nsq[None, :]


def _prep_z1(z1g, rgbg):
    return pl.pallas_call(
        _prep_z1_body,
        out_shape=(
            jax.ShapeDtypeStruct((_B, _PPAD, _ROWW), jnp.float32),
            jax.ShapeDtypeStruct((_B, _PPAD), jnp.float32),
        ),
    )(z1g, rgbg)


def _sc_main(table, z1flat, qidx, eucw):
    sbtot = _GROUPS // _SB
    sbbase = sbtot // _NW
    sbrem = sbtot % _NW
    mesh = plsc.VectorSubcoreMesh(core_axis_name="c", subcore_axis_name="s")

    @functools.partial(
        pl.kernel,
        mesh=mesh,
        out_type=(
            jax.ShapeDtypeStruct((_NPAIR * 16,), jnp.float32),
            jax.ShapeDtypeStruct((_NPAIR * 16,), jnp.float32),
        ),
        scratch_types=[
            pltpu.VMEM((2 * _SB, _CHUNK), jnp.int32),
            pltpu.VMEM((_CHUNK, _ROWW), jnp.float32),
            pltpu.VMEM((_CHUNK, _ROWW), jnp.float32),
            pltpu.VMEM((_SB * _ROWW,), jnp.float32),
            pltpu.VMEM((_SB * _NEG,), jnp.float32),
            pltpu.VMEM((_CHUNK * 16,), jnp.float32),
            pltpu.VMEM((_CHUNK * 16,), jnp.float32),
            pltpu.SemaphoreType.DMA,
            pltpu.SemaphoreType.DMA,
        ],
    )
    def k(table_hbm, z1_hbm, qidx_hbm, eucw_hbm, psum_hbm, head_hbm,
          idx_v, rows0_v, rows1_v, z1_v, ew_v, ps_v, hd_v, sem0, sem1):
        wid = lax.axis_index("s") * 2 + lax.axis_index("c")
        sb0 = wid * sbbase + jnp.minimum(wid, sbrem)
        nsb = sbbase + jnp.where(wid < sbrem, 1, 0)
        lane = lax.iota(jnp.int32, 16)

        def compute_chunk(rows_v, pbase, zc, zpart, ewbase):
            @functools.partial(plsc.parallel_loop, 0, _CHUNK // 16, unroll=2)
            def pair_body(blk):
                evec = ew_v[pl.ds(ewbase + blk * 16, 16)]
                for u in range(16):
                    j = blk * 16 + u
                    acc = rows_v[j, pl.ds(0, 16)] * zc[0]
                    for cc in range(1, _C // 16):
                        acc = acc + rows_v[j, pl.ds(cc * 16, 16)] * zc[cc]
                    ps_v[pl.ds(j * 16, 16)] = acc
                    head = rows_v[j, pl.ds(_C, 16)]
                    hd_v[pl.ds(j * 16, 16)] = (
                        head + zpart + jnp.where(lane == 4, evec[u], 0.0))
            pltpu.sync_copy(ps_v, psum_hbm.at[pl.ds(pbase * 16, _CHUNK * 16)])
            pltpu.sync_copy(hd_v, head_hbm.at[pl.ds(pbase * 16, _CHUNK * 16)])

        def sb_body(sb, carry):
            gbase = (sb0 + sb) * _SB
            pltpu.sync_copy(qidx_hbm.at[pl.ds(gbase * 2, 2 * _SB)], idx_v)
            pltpu.sync_copy(z1_hbm.at[pl.ds(gbase * _ROWW, _SB * _ROWW)], z1_v)
            pltpu.sync_copy(eucw_hbm.at[pl.ds(gbase * _NEG, _SB * _NEG)], ew_v)
            pltpu.async_copy(table_hbm.at[idx_v.at[0]], rows0_v, sem0)

            def g_body(s, carry2):
                g = gbase + s
                zoff = s * _ROWW
                zc = [z1_v[pl.ds(zoff + cc * 16, 16)] for cc in range(_C // 16)]
                zhead = z1_v[pl.ds(zoff + _C, 16)]
                zpart = jnp.zeros((16,), jnp.float32)
                for kk in range(4):
                    bc = jnp.take(zhead, jnp.full((16,), kk, jnp.int32))
                    zpart = zpart + jnp.where(lane == 5 + kk, bc, 0.0)
                pltpu.async_copy(table_hbm.at[idx_v.at[2 * s + 1]], rows1_v, sem1)
                pltpu.make_async_copy(
                    table_hbm.at[idx_v.at[0]], rows0_v, sem0).wait()
                compute_chunk(rows0_v, g * _NEG, zc, zpart, s * _NEG)

                @pl.when(s + 1 < _SB)
                def _():
                    pltpu.async_copy(
                        table_hbm.at[idx_v.at[2 * s + 2]], rows0_v, sem0)

                pltpu.make_async_copy(
                    table_hbm.at[idx_v.at[0]], rows1_v, sem1).wait()
                compute_chunk(rows1_v, g * _NEG + _CHUNK, zc, zpart,
                              s * _NEG + _CHUNK)
                return carry2

            lax.fori_loop(0, _SB, g_body, 0)
            return carry

        lax.fori_loop(0, nsb, sb_body, 0)

    return k(table, z1flat, qidx, eucw)


_ROWS2 = _NG * _NEG * 16 // 128


def _stage2_body(ps_ref, hd_ref, sel_ref, out_ref):
    b = pl.program_id(0)
    t = pl.program_id(1)
    dn = (((1,), (0,)), ((), ()))
    X = ps_ref[0]
    H = hd_ref[0]

    def mm(A, k):
        return lax.dot_general(A, sel_ref[k], dn,
                               preferred_element_type=jnp.float32)

    dot = mm(X, 0)
    n2 = mm(H, 1)
    ew = mm(H, 2)
    n1 = mm(H, 3)
    dr = mm(H, 4)
    dg = mm(H, 5)
    db = mm(H, 6)
    rgbsq = dr * dr + dg * dg + db * db
    w = ew + _RGBC * jnp.sqrt(rgbsq)
    denom = jnp.maximum(n1 * n2, _EPS)
    val = jnp.minimum(jnp.abs(dot * w / denom), 1.0)
    part = jnp.sum(val.reshape(_NG, _ROWS2 // _NG, 128), axis=0)

    @pl.when(jnp.logical_and(b == 0, t == 0))
    def _():
        out_ref[...] = jnp.zeros_like(out_ref)

    out_ref[pl.ds(b, 1)] += part[None]


def _stage2(psums, heads, sel):
    return pl.pallas_call(
        _stage2_body,
        grid=(_B, _NG),
        in_specs=[
            pl.BlockSpec((1, _ROWS2, 128), lambda b, t: (b * _NG + t, 0, 0)),
            pl.BlockSpec((1, _ROWS2, 128), lambda b, t: (b * _NG + t, 0, 0)),
            pl.BlockSpec((7, 128, 128), lambda b, t: (0, 0, 0)),
        ],
        out_specs=pl.BlockSpec((_B, _ROWS2 // _NG, 128), lambda b, t: (0, 0, 0)),
        out_shape=jax.ShapeDtypeStruct((_B, _ROWS2 // _NG, 128), jnp.float32),
    )(psums, heads, sel)


def _final_body(s_ref, nsq_ref, out_ref):
    s4 = s_ref[...].reshape(_B, 32, 8, 16)
    lane0 = (lax.broadcasted_iota(jnp.int32, (16,), 0) == 0).astype(jnp.float32)
    S = lax.dot_general(s4, lane0, (((3,), (0,)), ((), ()))).reshape(_B, _NEG)
    sneg = S * (1.0 / (_P * 2.0))
    nsq = nsq_ref[...]
    s0 = jnp.minimum(jnp.abs(nsq / jnp.maximum(nsq, _EPS)), 1.0)
    sim0 = jnp.sum(s0, axis=1) * (1.0 / _P)
    logp = jnp.clip(jnp.log(sim0), -100.0, None)
    log1m = jnp.clip(jnp.log(1.0 - sneg), -100.0, None)
    loss_b = -(logp + jnp.sum(log1m, axis=1)) * (1.0 / (_NEG + 1))
    loss = jnp.mean(loss_b)
    out2 = jnp.mean(sim0)
    out3 = jnp.sum(sneg) * (2.0 / (_NEG * _B))
    lane = lax.broadcasted_iota(jnp.int32, (1, 128), 1)
    out_ref[...] = (jnp.where(lane == 0, loss, 0.0)
                    + jnp.where(lane == 1, out2, 0.0)
                    + jnp.where(lane == 2, out3, 0.0))


def _final(S, nsq):
    return pl.pallas_call(
        _final_body,
        out_shape=jax.ShapeDtypeStruct((1, 128), jnp.float32),
    )(S, nsq)


def kernel(views_1, views_2, img):
    starts, qidx_np, eucw_np, sel_np = _host_constants()
    qidx = jnp.asarray(qidx_np)
    eucw = jnp.asarray(eucw_np)
    sel = jnp.asarray(sel_np)

    v2 = views_2.reshape(_B, _C, _PIX)
    img0 = img[0].reshape(3, _PIX)
    table = _prep_table(v2, img0)

    z1list, rgblist = [], []
    for b, (si, sj) in enumerate(starts):
        z1list.append(
            lax.slice(views_1, (b, 0, si, sj), (b + 1, _C, si + 209, sj + 209),
                      (1, 1, _GS, _GS)).reshape(_C, _P))
        rgblist.append(
            lax.slice(img, (0, 0, si, sj), (1, 3, si + 209, sj + 209),
                      (1, 1, _GS, _GS)).reshape(3, _P))
    z1g = jnp.pad(jnp.stack(z1list), ((0, 0), (0, 0), (0, _PPAD - _P)))
    rgbg = jnp.pad(jnp.stack(rgblist), ((0, 0), (0, 0), (0, _PPAD - _P)))
    z1tab, nsq = _prep_z1(z1g, rgbg)

    z1c = z1tab[:, :_P, :].reshape(_GROUPS, _ROWW)
    z1c = jnp.pad(z1c, ((0, _GPAD - _GROUPS), (0, 0)))
    psums, heads = _sc_main(table, z1c.reshape(-1), qidx, eucw)

    S = _stage2(psums.reshape(_GTILE, _ROWS2, 128),
                heads.reshape(_GTILE, _ROWS2, 128), sel)
    out = _final(S, nsq)
    return out[0, 0], out[0, 1], out[0, 2]
